# baseline (speedup 1.0000x reference)
import functools

import jax
import jax.numpy as jnp
from jax import lax
from jax.experimental import pallas as pl
from jax.experimental.pallas import tpu as pltpu
from jax.experimental.pallas import tpu_sc as plsc

_ATOM = 64
_BOND = 16
_B, _N, _E = 4, 512, 4096
_BP = 8
_NC, _NS = 2, 16
_EPT = _B * _E // (_NC * _NS)
_ROWS = _N * _BP
_ZR = _ROWS // _NS
_LANES = 128


def _prep_body(atom_ref, bond_ref, src_ref, tgt_ref, bt_ref,
               bmsg_ref, asum_ref, gsrc_ref, gtgt_ref):
    b = pl.program_id(0)
    bt16 = bt_ref[...].astype(jnp.bfloat16).astype(jnp.float32)
    w2 = jnp.sum(bt16, axis=-1)
    bond16 = bond_ref[0].astype(jnp.bfloat16).astype(jnp.float32)
    bmsg = jnp.dot(bond16, w2, preferred_element_type=jnp.float32,
                   precision=jax.lax.Precision.HIGHEST)
    bmsg_ref[0] = jnp.concatenate(
        [bmsg, jnp.zeros((_E, _LANES - _ATOM), jnp.float32)], axis=-1)
    asum_ref[0] = jnp.sum(atom_ref[0], axis=-1, keepdims=True)
    gsrc_ref[0] = src_ref[0] + b * _N
    gtgt_ref[0] = tgt_ref[0] * _BP + b


def _sc_scatter_body(bmsg_hbm, asum_hbm, gsrc_hbm, gtgt_hbm, zrows_hbm,
                     out_hbm, gsrc_v, gtgt_v, asum_v, s_v, msg_v, agg_sh):
    cid = lax.axis_index("c")
    sid = lax.axis_index("s")
    wid = cid * _NS + sid
    base = wid * _EPT
    pltpu.sync_copy(gsrc_hbm.at[pl.ds(base, _EPT)], gsrc_v)
    pltpu.sync_copy(gtgt_hbm.at[wid], gtgt_v)
    pltpu.sync_copy(asum_hbm, asum_v)
    pltpu.sync_copy(bmsg_hbm.at[pl.ds(base, _EPT)], msg_v)
    pltpu.sync_copy(zrows_hbm, agg_sh.at[pl.ds(sid * _ZR, _ZR)])

    def gather_body(i, carry):
        sl = pl.ds(i * 16, 16)
        s_v[sl] = plsc.load_gather(asum_v, [gsrc_v[sl]])
        return carry

    lax.fori_loop(0, _EPT // 16, gather_body, 0)

    def scale_body(i, carry):
        sv = s_v[pl.ds(i * 16, 16)]
        for j in range(16):
            e = i * 16 + j
            sc = sv[j]
            for k in range(_ATOM // 16):
                sl = pl.ds(k * 16, 16)
                msg_v[e, sl] = msg_v[e, sl] * sc
        return carry

    lax.fori_loop(0, _EPT // 16, scale_body, 0)

    plsc.subcore_barrier()
    for j in range(_EPT // 128):
        pltpu.sync_copy(msg_v.at[pl.ds(j * 128, 128)],
                        agg_sh.at[gtgt_v.at[j]], add=True)
    plsc.subcore_barrier()
    pltpu.sync_copy(agg_sh.at[pl.ds(sid * _ZR, _ZR)],
                    out_hbm.at[cid, pl.ds(sid * _ZR, _ZR)])


def _gru_body(atomT_ref, parts_ref, k_ref, r_ref, b_ref, out_ref, xmat_ref):
    a = atomT_ref[...]
    g = parts_ref[0, :, 0:_ATOM] + parts_ref[1, :, 0:_ATOM]
    xmat_ref[...] = (jnp.dot(a, k_ref[0:_ATOM, :])
                     + jnp.dot(g, k_ref[_ATOM:, :])
                     + b_ref[0:1, :])
    rmat = r_ref[...]
    br = b_ref[1:2, :]

    def step(t, h):
        xm = xmat_ref[pl.ds(t * _BP, _BP), :]
        hm = jnp.dot(h, rmat, preferred_element_type=jnp.float32) + br
        xz = xm[:, 0:_ATOM]
        xr = xm[:, _ATOM:2 * _ATOM]
        xh = xm[:, 2 * _ATOM:]
        rz = hm[:, 0:_ATOM]
        rr = hm[:, _ATOM:2 * _ATOM]
        rh = hm[:, 2 * _ATOM:]
        z = jax.nn.sigmoid(xz + rz)
        r = jax.nn.sigmoid(xr + rr)
        hcand = jnp.tanh(xh + r * rh)
        hn = z * h + (1.0 - z) * hcand
        out_ref[pl.ds(t * _BP, _BP), :] = hn
        return hn

    lax.fori_loop(0, _N, step, jnp.zeros((_BP, _ATOM), jnp.float32))


def kernel(atom_features, bond_features, connectivity, bond_transform,
           gru_kernel, gru_recurrent_kernel, gru_bias):
    src = connectivity[:, :, 0:1]
    tgt = connectivity[:, :, 1:2]

    bmsg, asum, gsrc, gtgt = pl.pallas_call(
        _prep_body,
        grid=(_B,),
        in_specs=[
            pl.BlockSpec((1, _N, _ATOM), lambda b: (b, 0, 0)),
            pl.BlockSpec((1, _E, _BOND), lambda b: (b, 0, 0)),
            pl.BlockSpec((1, _E, 1), lambda b: (b, 0, 0)),
            pl.BlockSpec((1, _E, 1), lambda b: (b, 0, 0)),
            pl.BlockSpec((_BOND, _ATOM, _ATOM), lambda b: (0, 0, 0)),
        ],
        out_specs=[
            pl.BlockSpec((1, _E, _LANES), lambda b: (b, 0, 0)),
            pl.BlockSpec((1, _N, 1), lambda b: (b, 0, 0)),
            pl.BlockSpec((1, _E, 1), lambda b: (b, 0, 0)),
            pl.BlockSpec((1, _E, 1), lambda b: (b, 0, 0)),
        ],
        out_shape=[
            jax.ShapeDtypeStruct((_B, _E, _LANES), jnp.float32),
            jax.ShapeDtypeStruct((_B, _N, 1), jnp.float32),
            jax.ShapeDtypeStruct((_B, _E, 1), jnp.int32),
            jax.ShapeDtypeStruct((_B, _E, 1), jnp.int32),
        ],
    )(atom_features, bond_features, src, tgt, bond_transform)

    sc_scatter = functools.partial(
        pl.kernel,
        out_type=jax.ShapeDtypeStruct((_NC, _ROWS, _LANES), jnp.float32),
        mesh=plsc.VectorSubcoreMesh(core_axis_name="c", subcore_axis_name="s",
                                    num_cores=_NC, num_subcores=_NS),
        compiler_params=pltpu.CompilerParams(needs_layout_passes=False),
        scratch_types=[
            pltpu.VMEM((_EPT,), jnp.int32),
            pltpu.VMEM((_EPT // 128, 128), jnp.int32),
            pltpu.VMEM((_B * _N,), jnp.float32),
            pltpu.VMEM((_EPT,), jnp.float32),
            pltpu.VMEM((_EPT, _LANES), jnp.float32),
            pltpu.VMEM_SHARED((_ROWS, _LANES), jnp.float32),
        ],
    )(_sc_scatter_body)

    parts = sc_scatter(
        bmsg.reshape(_B * _E, _LANES),
        asum.reshape(_B * _N),
        gsrc.reshape(_B * _E),
        gtgt.reshape(_NC * _NS, _EPT // 128, 128),
        jnp.zeros((_ZR, _LANES), jnp.float32),
    )

    atomT = jnp.zeros((_N, _BP, _ATOM), jnp.float32)
    atomT = atomT.at[:, :_B].set(jnp.swapaxes(atom_features, 0, 1))
    atomT2 = atomT.reshape(_ROWS, _ATOM)

    out2 = pl.pallas_call(
        _gru_body,
        out_shape=jax.ShapeDtypeStruct((_ROWS, _ATOM), jnp.float32),
        scratch_shapes=[pltpu.VMEM((_ROWS, 3 * _ATOM), jnp.float32)],
    )(atomT2, parts, gru_kernel, gru_recurrent_kernel, gru_bias)

    out = out2.reshape(_N, _BP, _ATOM)[:, :_B]
    return jnp.swapaxes(out, 0, 1)


# speedup vs baseline: 1.0211x; 1.0211x over previous
import functools

import jax
import jax.numpy as jnp
from jax import lax
from jax.experimental import pallas as pl
from jax.experimental.pallas import tpu as pltpu
from jax.experimental.pallas import tpu_sc as plsc

_ATOM = 64
_BOND = 16
_B, _N, _E = 4, 512, 4096
_BP = 8
_NC, _NS = 2, 16
_EPT = _B * _E // (_NC * _NS)
_ROWS = _N * _BP
_ZR = _ROWS // _NS
_LANES = 128


def _prep_body(atom_ref, bond_ref, src_ref, tgt_ref, bt_ref,
               bmsg_ref, asum_ref, gsrc_ref, gtgt_ref):
    b = pl.program_id(0)
    bt16 = bt_ref[...].astype(jnp.bfloat16).astype(jnp.float32)
    w2 = jnp.sum(bt16, axis=-1)
    bond16 = bond_ref[0].astype(jnp.bfloat16).astype(jnp.float32)
    bmsg = jnp.dot(bond16, w2, preferred_element_type=jnp.float32,
                   precision=jax.lax.Precision.HIGHEST)
    bmsg_ref[0] = jnp.concatenate(
        [bmsg, jnp.zeros((_E, _LANES - _ATOM), jnp.float32)], axis=-1)
    asum_ref[0] = jnp.sum(atom_ref[0], axis=-1, keepdims=True)
    gsrc_ref[0] = src_ref[0] + b * _N
    gtgt_ref[0] = tgt_ref[0] * _BP + b


def _sc_scatter_body(bmsg_hbm, asum_hbm, gsrc_hbm, gtgt_hbm, zrows_hbm,
                     out_hbm, gsrc_v, gtgt_v, asum_v, s_v, msg_v, agg_sh):
    cid = lax.axis_index("c")
    sid = lax.axis_index("s")
    wid = cid * _NS + sid
    base = wid * _EPT
    pltpu.sync_copy(gsrc_hbm.at[pl.ds(base, _EPT)], gsrc_v)
    pltpu.sync_copy(gtgt_hbm.at[wid], gtgt_v)
    pltpu.sync_copy(asum_hbm, asum_v)
    pltpu.sync_copy(bmsg_hbm.at[pl.ds(base, _EPT)], msg_v)
    pltpu.sync_copy(zrows_hbm, agg_sh.at[pl.ds(sid * _ZR, _ZR)])

    def gather_body(i, carry):
        sl = pl.ds(i * 16, 16)
        s_v[sl] = plsc.load_gather(asum_v, [gsrc_v[sl]])
        return carry

    lax.fori_loop(0, _EPT // 16, gather_body, 0)

    def scale_body(i, carry):
        sv = s_v[pl.ds(i * 16, 16)]
        for j in range(16):
            e = i * 16 + j
            sc = sv[j]
            for k in range(_ATOM // 16):
                sl = pl.ds(k * 16, 16)
                msg_v[e, sl] = msg_v[e, sl] * sc
        return carry

    lax.fori_loop(0, _EPT // 16, scale_body, 0)

    plsc.subcore_barrier()
    for j in range(_EPT // 128):
        pltpu.sync_copy(msg_v.at[pl.ds(j * 128, 128)],
                        agg_sh.at[gtgt_v.at[j]], add=True)
    plsc.subcore_barrier()
    pltpu.sync_copy(agg_sh.at[pl.ds(sid * _ZR, _ZR)],
                    out_hbm.at[cid, pl.ds(sid * _ZR, _ZR)])


def _gru_body(atomT_ref, parts_ref, k_ref, r_ref, b_ref, out_ref, xmat_ref):
    a = atomT_ref[...]
    g = parts_ref[0, :, 0:_ATOM] + parts_ref[1, :, 0:_ATOM]
    xmat_ref[...] = (jnp.dot(a, k_ref[0:_ATOM, :])
                     + jnp.dot(g, k_ref[_ATOM:, :])
                     + b_ref[0:1, :])
    rmat = r_ref[...]
    br = b_ref[1:2, :]

    _UNROLL = 8

    def step(i, h):
        for u in range(_UNROLL):
            xm = xmat_ref[pl.ds(i * (_BP * _UNROLL) + u * _BP, _BP), :]
            hm = jnp.dot(h, rmat, preferred_element_type=jnp.float32) + br
            xz = xm[:, 0:_ATOM]
            xr = xm[:, _ATOM:2 * _ATOM]
            xh = xm[:, 2 * _ATOM:]
            rz = hm[:, 0:_ATOM]
            rr = hm[:, _ATOM:2 * _ATOM]
            rh = hm[:, 2 * _ATOM:]
            z = jax.nn.sigmoid(xz + rz)
            r = jax.nn.sigmoid(xr + rr)
            hcand = jnp.tanh(xh + r * rh)
            h = z * h + (1.0 - z) * hcand
            out_ref[pl.ds(i * (_BP * _UNROLL) + u * _BP, _BP), :] = h
        return h

    lax.fori_loop(0, _N // _UNROLL, step, jnp.zeros((_BP, _ATOM), jnp.float32))


def kernel(atom_features, bond_features, connectivity, bond_transform,
           gru_kernel, gru_recurrent_kernel, gru_bias):
    src = connectivity[:, :, 0:1]
    tgt = connectivity[:, :, 1:2]

    bmsg, asum, gsrc, gtgt = pl.pallas_call(
        _prep_body,
        grid=(_B,),
        in_specs=[
            pl.BlockSpec((1, _N, _ATOM), lambda b: (b, 0, 0)),
            pl.BlockSpec((1, _E, _BOND), lambda b: (b, 0, 0)),
            pl.BlockSpec((1, _E, 1), lambda b: (b, 0, 0)),
            pl.BlockSpec((1, _E, 1), lambda b: (b, 0, 0)),
            pl.BlockSpec((_BOND, _ATOM, _ATOM), lambda b: (0, 0, 0)),
        ],
        out_specs=[
            pl.BlockSpec((1, _E, _LANES), lambda b: (b, 0, 0)),
            pl.BlockSpec((1, _N, 1), lambda b: (b, 0, 0)),
            pl.BlockSpec((1, _E, 1), lambda b: (b, 0, 0)),
            pl.BlockSpec((1, _E, 1), lambda b: (b, 0, 0)),
        ],
        out_shape=[
            jax.ShapeDtypeStruct((_B, _E, _LANES), jnp.float32),
            jax.ShapeDtypeStruct((_B, _N, 1), jnp.float32),
            jax.ShapeDtypeStruct((_B, _E, 1), jnp.int32),
            jax.ShapeDtypeStruct((_B, _E, 1), jnp.int32),
        ],
    )(atom_features, bond_features, src, tgt, bond_transform)

    sc_scatter = functools.partial(
        pl.kernel,
        out_type=jax.ShapeDtypeStruct((_NC, _ROWS, _LANES), jnp.float32),
        mesh=plsc.VectorSubcoreMesh(core_axis_name="c", subcore_axis_name="s",
                                    num_cores=_NC, num_subcores=_NS),
        compiler_params=pltpu.CompilerParams(needs_layout_passes=False),
        scratch_types=[
            pltpu.VMEM((_EPT,), jnp.int32),
            pltpu.VMEM((_EPT // 128, 128), jnp.int32),
            pltpu.VMEM((_B * _N,), jnp.float32),
            pltpu.VMEM((_EPT,), jnp.float32),
            pltpu.VMEM((_EPT, _LANES), jnp.float32),
            pltpu.VMEM_SHARED((_ROWS, _LANES), jnp.float32),
        ],
    )(_sc_scatter_body)

    parts = sc_scatter(
        bmsg.reshape(_B * _E, _LANES),
        asum.reshape(_B * _N),
        gsrc.reshape(_B * _E),
        gtgt.reshape(_NC * _NS, _EPT // 128, 128),
        jnp.zeros((_ZR, _LANES), jnp.float32),
    )

    atomT = jnp.zeros((_N, _BP, _ATOM), jnp.float32)
    atomT = atomT.at[:, :_B].set(jnp.swapaxes(atom_features, 0, 1))
    atomT2 = atomT.reshape(_ROWS, _ATOM)

    out2 = pl.pallas_call(
        _gru_body,
        out_shape=jax.ShapeDtypeStruct((_ROWS, _ATOM), jnp.float32),
        scratch_shapes=[pltpu.VMEM((_ROWS, 3 * _ATOM), jnp.float32)],
    )(atomT2, parts, gru_kernel, gru_recurrent_kernel, gru_bias)

    out = out2.reshape(_N, _BP, _ATOM)[:, :_B]
    return jnp.swapaxes(out, 0, 1)
